# baseline (speedup 1.0000x reference)
import jax
import jax.numpy as jnp
from jax.experimental import pallas as pl
from jax.experimental.pallas import tpu as pltpu

HIDDEN = 4096
N_EXPERTS = 64
TOP_K = 8
BT = 1024


def _gate_block(x_ref, w_ref, vals_ref, idx_ref):
    logits = jax.lax.dot_general(
        x_ref[...].astype(jnp.bfloat16), w_ref[...].astype(jnp.bfloat16),
        dimension_numbers=(((1,), (1,)), ((), ())),
        preferred_element_type=jnp.float32,
    )
    m = jnp.max(logits, axis=1, keepdims=True)
    e = jnp.exp(logits - m)
    s = jnp.sum(e, axis=1, keepdims=True)

    iota = jax.lax.broadcasted_iota(jnp.int32, e.shape, 1).astype(jnp.float32)
    for k in range(TOP_K):
        v = jnp.max(e, axis=1, keepdims=True)
        cand = jnp.where(e == v, iota, float(N_EXPERTS))
        ix = jnp.min(cand, axis=1, keepdims=True)
        vals_ref[:, k] = (v / s)[:, 0]
        idx_ref[:, k] = ix[:, 0].astype(jnp.int32)
        e = jnp.where(iota == ix, -1.0, e)


@jax.jit
def kernel(x, W):
    tokens = x.shape[0]
    grid = (pl.cdiv(tokens, BT),)
    vals, idx = pl.pallas_call(
        _gate_block,
        grid=grid,
        in_specs=[
            pl.BlockSpec((BT, HIDDEN), lambda i: (i, 0)),
            pl.BlockSpec((N_EXPERTS, HIDDEN), lambda i: (0, 0)),
        ],
        out_specs=[
            pl.BlockSpec((BT, TOP_K), lambda i: (i, 0)),
            pl.BlockSpec((BT, TOP_K), lambda i: (i, 0)),
        ],
        out_shape=[
            jax.ShapeDtypeStruct((tokens, TOP_K), jnp.float32),
            jax.ShapeDtypeStruct((tokens, TOP_K), jnp.int32),
        ],
        compiler_params=pltpu.CompilerParams(
            dimension_semantics=("arbitrary",),
            vmem_limit_bytes=100 * 1024 * 1024,
        ),
    )(x, W)
    return vals, idx


# speedup vs baseline: 1.0017x; 1.0017x over previous
import jax
import jax.numpy as jnp
from jax.experimental import pallas as pl
from jax.experimental.pallas import tpu as pltpu

HIDDEN = 4096
HH = HIDDEN // 2
N_EXPERTS = 64
TOP_K = 8
BT = 1024


def _gate_block(xa_ref, xb_ref, w_ref, vals_ref, idx_ref):
    wb = w_ref[...].astype(jnp.bfloat16)
    logits = jax.lax.dot_general(
        xa_ref[...].astype(jnp.bfloat16), wb[:, :HH],
        dimension_numbers=(((1,), (1,)), ((), ())),
        preferred_element_type=jnp.float32,
    ) + jax.lax.dot_general(
        xb_ref[...].astype(jnp.bfloat16), wb[:, HH:],
        dimension_numbers=(((1,), (1,)), ((), ())),
        preferred_element_type=jnp.float32,
    )
    m = jnp.max(logits, axis=1, keepdims=True)
    e = jnp.exp(logits - m)
    s = jnp.sum(e, axis=1, keepdims=True)

    iota = jax.lax.broadcasted_iota(jnp.int32, e.shape, 1).astype(jnp.float32)
    for k in range(TOP_K):
        v = jnp.max(e, axis=1, keepdims=True)
        cand = jnp.where(e == v, iota, float(N_EXPERTS))
        ix = jnp.min(cand, axis=1, keepdims=True)
        vals_ref[:, k] = (v / s)[:, 0]
        idx_ref[:, k] = ix[:, 0].astype(jnp.int32)
        e = jnp.where(iota == ix, -1.0, e)


@jax.jit
def kernel(x, W):
    tokens = x.shape[0]
    grid = (pl.cdiv(tokens, BT),)
    vals, idx = pl.pallas_call(
        _gate_block,
        grid=grid,
        in_specs=[
            pl.BlockSpec((BT, HH), lambda i: (i, 0)),
            pl.BlockSpec((BT, HH), lambda i: (i, 1)),
            pl.BlockSpec((N_EXPERTS, HIDDEN), lambda i: (0, 0)),
        ],
        out_specs=[
            pl.BlockSpec((BT, TOP_K), lambda i: (i, 0)),
            pl.BlockSpec((BT, TOP_K), lambda i: (i, 0)),
        ],
        out_shape=[
            jax.ShapeDtypeStruct((tokens, TOP_K), jnp.float32),
            jax.ShapeDtypeStruct((tokens, TOP_K), jnp.int32),
        ],
        compiler_params=pltpu.CompilerParams(
            dimension_semantics=("arbitrary",),
            vmem_limit_bytes=100 * 1024 * 1024,
        ),
    )(x, x, W)
    return vals, idx


# speedup vs baseline: 1.0034x; 1.0017x over previous
import jax
import jax.numpy as jnp
from jax.experimental import pallas as pl
from jax.experimental.pallas import tpu as pltpu

HIDDEN = 4096
N_EXPERTS = 64
TOP_K = 8
BT = 1024


def _gate_block(x_ref, w_ref, vals_ref, idx_ref):
    logits = jax.lax.dot_general(
        x_ref[...].astype(jnp.bfloat16), w_ref[...].astype(jnp.bfloat16),
        dimension_numbers=(((1,), (1,)), ((), ())),
        preferred_element_type=jnp.float32,
    )
    m = jnp.max(logits, axis=1, keepdims=True)
    e = jnp.exp(logits - m)
    s = jnp.sum(e, axis=1, keepdims=True)

    iota = jax.lax.broadcasted_iota(jnp.int32, e.shape, 1).astype(jnp.float32)
    for k in range(TOP_K):
        v = jnp.max(e, axis=1, keepdims=True)
        cand = jnp.where(e == v, iota, float(N_EXPERTS))
        ix = jnp.min(cand, axis=1, keepdims=True)
        vals_ref[:, k] = (v / s)[:, 0]
        idx_ref[:, k] = ix[:, 0].astype(jnp.int32)
        e = jnp.where(iota == ix, -1.0, e)


@jax.jit
def kernel(x, W):
    tokens = x.shape[0]
    grid = (pl.cdiv(tokens, BT),)
    vals, idx = pl.pallas_call(
        _gate_block,
        grid=grid,
        in_specs=[
            pl.BlockSpec((BT, HIDDEN), lambda i: (i, 0)),
            pl.BlockSpec((N_EXPERTS, HIDDEN), lambda i: (0, 0)),
        ],
        out_specs=[
            pl.BlockSpec((BT, TOP_K), lambda i: (i, 0)),
            pl.BlockSpec((BT, TOP_K), lambda i: (i, 0)),
        ],
        out_shape=[
            jax.ShapeDtypeStruct((tokens, TOP_K), jnp.float32),
            jax.ShapeDtypeStruct((tokens, TOP_K), jnp.int32),
        ],
        compiler_params=pltpu.CompilerParams(
            dimension_semantics=("arbitrary",),
            vmem_limit_bytes=100 * 1024 * 1024,
        ),
    )(x, W)
    return vals, idx
